# baseline (speedup 1.0000x reference)
import functools
import jax
import jax.numpy as jnp
from jax import lax
from jax.experimental import pallas as pl
from jax.experimental.pallas import tpu as pltpu
from jax.experimental.pallas import tpu_sc as plsc

EMBED = 128
BATCH = 16384
NC = 2
NS = 16
NW = NC * NS
ROWS_PER_W = BATCH // NW
CH = 64
NCHUNK = ROWS_PER_W // CH
NBUF = 4
LANES = 16
KREG = EMBED // LANES
_DNUMS = lax.GatherDimensionNumbers(
    offset_dims=(), collapsed_slice_dims=(0,), start_index_map=(0,))


def _rsqrt_nr(v):
    i = lax.bitcast_convert_type(v, jnp.int32)
    i = jnp.int32(0x5F3759DF) - (i >> 1)
    y = lax.bitcast_convert_type(i, jnp.float32)
    vh = v * jnp.float32(-0.5)
    for _ in range(2):
        y = y * (jnp.float32(1.5) + vh * y * y)
    return y


def _body(z_hbm, y_hbm, table_hbm, out_hbm, idx_v, gbuf, obuf,
          zs0, zs1, zs2, zs3, gs0, gs1, gs2, gs3, ws0, ws1, ws2, ws3):
    wid = lax.axis_index("s") * NC + lax.axis_index("c")
    base = wid * ROWS_PER_W
    zsems = (zs0, zs1, zs2, zs3)
    gsems = (gs0, gs1, gs2, gs3)
    wsems = (ws0, ws1, ws2, ws3)

    zcps = [None] * NCHUNK
    gcps = [None] * NCHUNK
    wcps = [None] * NCHUNK

    def start_z(s):
        b = s % NBUF
        zcps[s] = pltpu.make_async_copy(
            z_hbm.at[pl.ds(base + s * CH, CH)],
            obuf.at[b, :, pl.ds(0, EMBED)],
            zsems[b])
        zcps[s].start()

    def start_g(s):
        b = s % NBUF
        gcps[s] = pltpu.make_async_copy(
            table_hbm.at[idx_v.at[pl.ds(s * CH, CH)]], gbuf.at[b], gsems[b])
        gcps[s].start()

    def start(s):
        if s >= NBUF:
            wcps[s - NBUF].wait()
        start_z(s)
        start_g(s)

    icp = pltpu.make_async_copy(
        y_hbm.at[pl.ds(base, ROWS_PER_W)], idx_v, gsems[NBUF - 1])
    icp.start()
    for s in range(NBUF):
        start_z(s)
    icp.wait()
    for s in range(NBUF):
        start_g(s)

    def normalize(b):
        @plsc.parallel_loop(0, CH, step=1, unroll=4)
        def _(r):
            vs = []
            acc = jnp.zeros((LANES,), jnp.float32)
            for k in range(KREG):
                v = gbuf[b, r, pl.ds(k * LANES, LANES)]
                vs.append(v)
                acc = acc + v * v
            iota = lax.iota(jnp.int32, LANES)
            for d in (8, 4, 2, 1):
                perm = lax.gather(
                    acc, (iota ^ d)[:, None], _DNUMS, slice_sizes=(1,),
                    mode=lax.GatherScatterMode.PROMISE_IN_BOUNDS)
                acc = acc + perm
            scale = _rsqrt_nr(acc)
            for k in range(KREG):
                obuf[b, r, pl.ds(EMBED + k * LANES, LANES)] = vs[k] * scale

    for s in range(NCHUNK):
        b = s % NBUF
        gcps[s].wait()
        normalize(b)
        zcps[s].wait()
        wcps[s] = pltpu.make_async_copy(
            obuf.at[b], out_hbm.at[pl.ds(base + s * CH, CH)], wsems[b])
        wcps[s].start()
        if s + NBUF < NCHUNK:
            start(s + NBUF)
    for s in range(NCHUNK - NBUF, NCHUNK):
        wcps[s].wait()


@jax.jit
def _launch(z, y3, table):
    mesh = plsc.VectorSubcoreMesh(core_axis_name="c", subcore_axis_name="s")
    f = functools.partial(
        pl.kernel,
        mesh=mesh,
        out_type=jax.ShapeDtypeStruct((BATCH, 2 * EMBED), jnp.float32),
        scratch_types=[
            pltpu.VMEM((ROWS_PER_W,), jnp.int32),
            pltpu.VMEM((NBUF, CH, EMBED), jnp.float32),
            pltpu.VMEM((NBUF, CH, 2 * EMBED), jnp.float32),
        ] + [pltpu.SemaphoreType.DMA] * (3 * NBUF),
    )(_body)
    return f(z, y3, table)


def kernel(z, y, table):
    return _launch(z, y.astype(jnp.int32), table)


# speedup vs baseline: 1.0174x; 1.0174x over previous
import functools
import jax
import jax.numpy as jnp
from jax import lax
from jax.experimental import pallas as pl
from jax.experimental.pallas import tpu as pltpu
from jax.experimental.pallas import tpu_sc as plsc

EMBED = 128
BATCH = 16384
NC = 2
NS = 16
NW = NC * NS
ROWS_PER_W = BATCH // NW
CH = 64
NCHUNK = ROWS_PER_W // CH
NBUF = 4
LANES = 16
KREG = EMBED // LANES
_DNUMS = lax.GatherDimensionNumbers(
    offset_dims=(), collapsed_slice_dims=(0,), start_index_map=(0,))


def _rsqrt_nr(v):
    i = lax.bitcast_convert_type(v, jnp.int32)
    i = jnp.int32(0x5F3759DF) - (i >> 1)
    y = lax.bitcast_convert_type(i, jnp.float32)
    vh = v * jnp.float32(-0.5)
    for _ in range(2):
        y = y * (jnp.float32(1.5) + vh * y * y)
    return y


def _body(z_hbm, y_hbm, table_hbm, out_hbm, idx_v, gbuf, obuf,
          zs0, zs1, zs2, zs3, gs0, gs1, gs2, gs3, ws0, ws1, ws2, ws3):
    wid = lax.axis_index("s") * NC + lax.axis_index("c")
    base = wid * ROWS_PER_W
    zsems = (zs0, zs1, zs2, zs3)
    gsems = (gs0, gs1, gs2, gs3)
    wsems = (ws0, ws1, ws2, ws3)

    pltpu.sync_copy(y_hbm.at[pl.ds(base, ROWS_PER_W)], idx_v)

    zcps = [None] * NCHUNK
    gcps = [None] * NCHUNK
    wcps = [None] * NCHUNK

    def start(s):
        b = s % NBUF
        if s >= NBUF:
            wcps[s - NBUF].wait()
        row0 = base + s * CH
        zcps[s] = pltpu.make_async_copy(
            z_hbm.at[pl.ds(row0, CH)],
            obuf.at[b, :, pl.ds(0, EMBED)],
            zsems[b])
        zcps[s].start()
        gcps[s] = pltpu.make_async_copy(
            table_hbm.at[idx_v.at[pl.ds(s * CH, CH)]], gbuf.at[b], gsems[b])
        gcps[s].start()

    def normalize(b):
        @plsc.parallel_loop(0, CH, step=1, unroll=4)
        def _(r):
            vs = []
            acc = jnp.zeros((LANES,), jnp.float32)
            for k in range(KREG):
                v = gbuf[b, r, pl.ds(k * LANES, LANES)]
                vs.append(v)
                acc = acc + v * v
            iota = lax.iota(jnp.int32, LANES)
            for d in (8, 4, 2, 1):
                perm = lax.gather(
                    acc, (iota ^ d)[:, None], _DNUMS, slice_sizes=(1,),
                    mode=lax.GatherScatterMode.PROMISE_IN_BOUNDS)
                acc = acc + perm
            scale = _rsqrt_nr(acc)
            for k in range(KREG):
                obuf[b, r, pl.ds(EMBED + k * LANES, LANES)] = vs[k] * scale

    for s in range(NBUF):
        start(s)
    for s in range(NCHUNK):
        b = s % NBUF
        gcps[s].wait()
        normalize(b)
        zcps[s].wait()
        wcps[s] = pltpu.make_async_copy(
            obuf.at[b], out_hbm.at[pl.ds(base + s * CH, CH)], wsems[b])
        wcps[s].start()
        if s + NBUF < NCHUNK:
            start(s + NBUF)
    for s in range(NCHUNK - NBUF, NCHUNK):
        wcps[s].wait()


@jax.jit
def _launch(z, y3, table):
    mesh = plsc.VectorSubcoreMesh(core_axis_name="c", subcore_axis_name="s")
    f = functools.partial(
        pl.kernel,
        mesh=mesh,
        out_type=jax.ShapeDtypeStruct((BATCH, 2 * EMBED), jnp.float32),
        scratch_types=[
            pltpu.VMEM((ROWS_PER_W,), jnp.int32),
            pltpu.VMEM((NBUF, CH, EMBED), jnp.float32),
            pltpu.VMEM((NBUF, CH, 2 * EMBED), jnp.float32),
        ] + [pltpu.SemaphoreType.DMA] * (3 * NBUF),
    )(_body)
    return f(z, y3, table)


def kernel(z, y, table):
    return _launch(z, y.astype(jnp.int32), table)
